# baseline (speedup 1.0000x reference)
import functools

import jax
import jax.numpy as jnp
from jax import lax
from jax.experimental import pallas as pl
from jax.experimental.pallas import tpu as pltpu
from jax.experimental.pallas import tpu_sc as plsc

B, S, D, H, E, K = 1, 2048, 1024, 4096, 8, 2
T = B * S
EP = 128
BM = 256
NT = (K * T) // BM + E
L = NT * BM
NHB = 4
HB = H // NHB

NC, NS = 2, 16
NW = NC * NS


def _shift_down(a, sh):
    return jnp.concatenate(
        [jnp.zeros((sh,) + a.shape[1:], a.dtype), a[:-sh]], axis=0)


def _router_kernel(x_ref, w_ref, b_ref, eps_ref, g_ref, dest_ref, cnt_ref):
    x = x_ref[...]
    w = w_ref[...]
    res = lax.dot_general(
        x.astype(jnp.bfloat16), w.astype(jnp.bfloat16),
        (((1,), (0,)), ((), ())),
        preferred_element_type=jnp.float32,
    )
    b = b_ref[0:1, :]
    logits = res[:, :EP] + b[:, :EP]
    nlogits = res[:, EP:] + b[:, EP:]
    sp = jnp.logaddexp(nlogits, 0.0)
    noisy = logits + eps_ref[...] * sp
    col = lax.broadcasted_iota(jnp.int32, (T, EP), 1)
    neg = jnp.float32(-jnp.inf)
    noisy = jnp.where(col < E, noisy, neg)
    m1 = jnp.max(noisy, axis=1, keepdims=True)
    idx1 = jnp.min(jnp.where(noisy == m1, col, EP), axis=1, keepdims=True)
    v2 = jnp.where(col == idx1, neg, noisy)
    m2 = jnp.max(v2, axis=1, keepdims=True)
    idx2 = jnp.min(jnp.where(v2 == m2, col, EP), axis=1, keepdims=True)
    em2 = jnp.exp(m2 - m1)
    denom = 1.0 + em2
    g_ref[...] = jnp.where(
        col == 0, 1.0 / denom, jnp.where(col == 1, em2 / denom, 0.0))

    sel = ((col == idx1) | (col == idx2)).astype(jnp.int32)
    counts = jnp.sum(sel, axis=0, keepdims=True)
    pc = ((counts + BM - 1) // BM) * BM
    col1 = lax.broadcasted_iota(jnp.int32, (1, EP), 1)
    p = pc
    for sh in (1, 2, 4):
        rolled = jnp.concatenate(
            [jnp.zeros((1, sh), jnp.int32), p[:, :EP - sh]], axis=1)
        p = p + jnp.where(col1 >= sh, rolled, 0)
    off = p - pc
    inc = sel
    sh = 1
    while sh < T:
        inc = inc + _shift_down(inc, sh)
        sh *= 2
    rank = inc - sel
    dest = off + rank
    d1 = jnp.sum(jnp.where(col == idx1, dest, 0), axis=1, keepdims=True)
    d2 = jnp.sum(jnp.where(col == idx2, dest, 0), axis=1, keepdims=True)
    dest_ref[...] = jnp.where(col == 0, d1, jnp.where(col == 1, d2, 0))
    cnt_ref[...] = jnp.broadcast_to(counts, (8, EP))


@functools.cache
def _sc_mesh():
    return plsc.VectorSubcoreMesh(core_axis_name="c", subcore_axis_name="s")


def _sc_scatter(idx_hbm, tok_hbm, gv_hbm, st_hbm, sg_hbm, idx_v, tok_v, gv_v):
    per = (K * T) // NW
    wid = lax.axis_index("s") * NC + lax.axis_index("c")
    base = wid * per
    pltpu.sync_copy(idx_hbm.at[pl.ds(base, per)], idx_v)
    pltpu.sync_copy(tok_hbm.at[pl.ds(base, per)], tok_v)
    pltpu.sync_copy(gv_hbm.at[pl.ds(base, per)], gv_v)
    pltpu.sync_copy(tok_v, st_hbm.at[idx_v])
    pltpu.sync_copy(gv_v, sg_hbm.at[idx_v])


def _sc_gather(x_hbm, st_hbm, xs_hbm, idx_v, rows_v, sem):
    chunk = rows_v.shape[0]
    per = L // NW
    wid = lax.axis_index("s") * NC + lax.axis_index("c")
    base = wid * per

    @pl.loop(0, per // chunk)
    def _(c):
        b = base + c * chunk
        pltpu.sync_copy(st_hbm.at[pl.ds(b, chunk)], idx_v)

        @pl.loop(0, chunk // 16)
        def _(i):
            sl = pl.ds(i * 16, 16)
            v = idx_v.at[sl][...]
            idx_v.at[sl][...] = jnp.minimum(jnp.maximum(v, 0), T - 1)

        pltpu.async_copy(x_hbm.at[idx_v], rows_v, sem).wait()
        pltpu.sync_copy(rows_v, xs_hbm.at[pl.ds(b, chunk)])


def _sc_combine(os_hbm, d0_hbm, d1_hbm, out_hbm, idx_v, r0_v, r1_v, sem):
    chunk = r0_v.shape[0]
    per = T // NW
    wid = lax.axis_index("s") * NC + lax.axis_index("c")
    base = wid * per

    @pl.loop(0, per // chunk)
    def _(c):
        b = base + c * chunk
        pltpu.sync_copy(d0_hbm.at[pl.ds(b, chunk)], idx_v)
        pltpu.async_copy(os_hbm.at[idx_v], r0_v, sem).wait()
        pltpu.sync_copy(d1_hbm.at[pl.ds(b, chunk)], idx_v)
        pltpu.async_copy(os_hbm.at[idx_v], r1_v, sem).wait()

        @pl.loop(0, chunk)
        def _(r):
            @pl.loop(0, D // 16)
            def _(j):
                sl = (pl.ds(r, 1), pl.ds(j * 16, 16))
                r0_v.at[sl][...] = r0_v.at[sl][...] + r1_v.at[sl][...]

        pltpu.sync_copy(r0_v, out_hbm.at[pl.ds(b, chunk)])


def _gffn_kernel(se_ref, nu_ref, xs_ref, w1_ref, b1_ref, w2_ref, b2_ref,
                 g_ref, o_ref, acc_ref):
    hb = pl.program_id(0)
    s = pl.program_id(1)

    @pl.when(s < nu_ref[0])
    def _():
        xb = xs_ref[...].astype(jnp.bfloat16)
        w1 = w1_ref[0].astype(jnp.bfloat16)
        h = lax.dot_general(
            xb, w1, (((1,), (0,)), ((), ())),
            preferred_element_type=jnp.float32)
        h = h + b1_ref[0]
        h = 0.5 * h * (1.0 + lax.erf(h * 0.7071067811865476))
        w2 = w2_ref[0].astype(jnp.bfloat16)
        po = lax.dot_general(
            h.astype(jnp.bfloat16), w2, (((1,), (0,)), ((), ())),
            preferred_element_type=jnp.float32)
        g = g_ref[...]
        contrib = g * po
        row = pl.ds(s * BM, BM)

        @pl.when(hb == 0)
        def _():
            acc_ref[row, :] = contrib + g * b2_ref[0]

        @pl.when(hb > 0)
        def _():
            acc_ref[row, :] += contrib

        @pl.when(hb == NHB - 1)
        def _():
            o_ref[...] = acc_ref[row, :]


def kernel(x, W_route, b_route, W_noise, b_noise, fc1_w, fc1_b, fc2_w, fc2_b):
    i32 = jnp.int32
    x2 = x.reshape(T, D)
    w = jnp.zeros((D, 2 * EP), jnp.float32)
    w = w.at[:, :E].set(W_route).at[:, EP:EP + E].set(W_noise)
    bvec = jnp.zeros((2 * EP,), jnp.float32)
    bvec = bvec.at[:E].set(b_route).at[EP:EP + E].set(b_noise)
    bvec = jnp.broadcast_to(bvec[None, :], (8, 2 * EP))
    eps = jax.random.normal(jax.random.key(42), (B, S, E), dtype=jnp.float32)
    eps_p = jnp.zeros((T, EP), jnp.float32).at[:, :E].set(eps.reshape(T, E))

    g_tk, dest_tk, cnt = pl.pallas_call(
        _router_kernel,
        out_shape=(
            jax.ShapeDtypeStruct((T, EP), jnp.float32),
            jax.ShapeDtypeStruct((T, EP), i32),
            jax.ShapeDtypeStruct((8, EP), i32),
        ),
    )(x2, w, bvec, eps_p)

    counts8 = cnt[0, :E]
    tiles = (counts8 + BM - 1) // BM
    tstart = jnp.concatenate([jnp.zeros((1,), i32), jnp.cumsum(tiles)])
    nused = tstart[E:E + 1].astype(i32)
    s_iota = jnp.arange(NT, dtype=i32)
    step_expert = jnp.minimum(
        jnp.sum((s_iota[:, None] >= tstart[None, 1:]).astype(i32), axis=1),
        E - 1).astype(i32)

    dest2 = dest_tk[:, :K]
    scatter_idx = dest2.reshape(K * T)
    gate_vals = g_tk[:, :K].reshape(K * T)
    d0 = dest2[:, 0]
    d1 = dest2[:, 1]
    tokvals = jnp.arange(K * T, dtype=i32) // K

    sorted_tok, sorted_gate = pl.kernel(
        _sc_scatter,
        out_type=(
            jax.ShapeDtypeStruct((L,), i32),
            jax.ShapeDtypeStruct((L,), jnp.float32),
        ),
        mesh=_sc_mesh(),
        scratch_types=[
            pltpu.VMEM(((K * T) // NW,), i32),
            pltpu.VMEM(((K * T) // NW,), i32),
            pltpu.VMEM(((K * T) // NW,), jnp.float32),
        ],
    )(scatter_idx, tokvals, gate_vals)

    x_sorted = pl.kernel(
        _sc_gather,
        out_type=jax.ShapeDtypeStruct((L, D), jnp.float32),
        mesh=_sc_mesh(),
        scratch_types=[
            pltpu.VMEM((64,), i32),
            pltpu.VMEM((64, D), jnp.float32),
            pltpu.SemaphoreType.DMA,
        ],
    )(x2, sorted_tok)

    o_sorted = pl.pallas_call(
        _gffn_kernel,
        grid_spec=pltpu.PrefetchScalarGridSpec(
            num_scalar_prefetch=2,
            grid=(NHB, NT),
            in_specs=[
                pl.BlockSpec(
                    (BM, D),
                    lambda hb, s, se, nu: (jnp.minimum(s, nu[0] - 1), 0)),
                pl.BlockSpec(
                    (1, D, HB),
                    lambda hb, s, se, nu:
                    (se[jnp.minimum(s, nu[0] - 1)], 0, hb)),
                pl.BlockSpec(
                    (1, 1, HB),
                    lambda hb, s, se, nu:
                    (se[jnp.minimum(s, nu[0] - 1)], 0, hb)),
                pl.BlockSpec(
                    (1, HB, D),
                    lambda hb, s, se, nu:
                    (se[jnp.minimum(s, nu[0] - 1)], hb, 0)),
                pl.BlockSpec(
                    (1, 1, D),
                    lambda hb, s, se, nu:
                    (se[jnp.minimum(s, nu[0] - 1)], 0, 0)),
                pl.BlockSpec(
                    (BM, 1),
                    lambda hb, s, se, nu: (jnp.minimum(s, nu[0] - 1), 0)),
            ],
            out_specs=pl.BlockSpec(
                (BM, D),
                lambda hb, s, se, nu: (jnp.where(hb == NHB - 1, s, 0), 0)),
            scratch_shapes=[pltpu.VMEM((L, D), jnp.float32)],
        ),
        out_shape=jax.ShapeDtypeStruct((L, D), jnp.float32),
    )(step_expert, nused, x_sorted, fc1_w, fc1_b.reshape(E, 1, H),
      fc2_w, fc2_b.reshape(E, 1, D), sorted_gate.reshape(L, 1))

    out = pl.kernel(
        _sc_combine,
        out_type=jax.ShapeDtypeStruct((T, D), jnp.float32),
        mesh=_sc_mesh(),
        scratch_types=[
            pltpu.VMEM((32,), i32),
            pltpu.VMEM((32, D), jnp.float32),
            pltpu.VMEM((32, D), jnp.float32),
            pltpu.SemaphoreType.DMA,
        ],
    )(o_sorted, d0, d1)

    return out.reshape(B, S, D)


# speedup vs baseline: 1.2363x; 1.2363x over previous
import functools

import jax
import jax.numpy as jnp
from jax import lax
from jax.experimental import pallas as pl
from jax.experimental.pallas import tpu as pltpu
from jax.experimental.pallas import tpu_sc as plsc

B, S, D, H, E, K = 1, 2048, 1024, 4096, 8, 2
T = B * S
EP = 128
BM = 256
NT = (K * T) // BM + E
L = NT * BM
NHB = 8
HB = H // NHB

NC, NS = 2, 16
NW = NC * NS


def _shift_down(a, sh):
    return jnp.concatenate(
        [jnp.zeros((sh,) + a.shape[1:], a.dtype), a[:-sh]], axis=0)


def _router_kernel(x_ref, w_ref, b_ref, eps_ref, g_ref, dest_ref, cnt_ref):
    x = x_ref[...]
    w = w_ref[...]
    res = lax.dot_general(
        x.astype(jnp.bfloat16), w.astype(jnp.bfloat16),
        (((1,), (0,)), ((), ())),
        preferred_element_type=jnp.float32,
    )
    b = b_ref[0:1, :]
    logits = res[:, :EP] + b[:, :EP]
    nlogits = res[:, EP:] + b[:, EP:]
    sp = jnp.logaddexp(nlogits, 0.0)
    noisy = logits + eps_ref[...] * sp
    col = lax.broadcasted_iota(jnp.int32, (T, EP), 1)
    neg = jnp.float32(-jnp.inf)
    noisy = jnp.where(col < E, noisy, neg)
    m1 = jnp.max(noisy, axis=1, keepdims=True)
    idx1 = jnp.min(jnp.where(noisy == m1, col, EP), axis=1, keepdims=True)
    v2 = jnp.where(col == idx1, neg, noisy)
    m2 = jnp.max(v2, axis=1, keepdims=True)
    idx2 = jnp.min(jnp.where(v2 == m2, col, EP), axis=1, keepdims=True)
    em2 = jnp.exp(m2 - m1)
    denom = 1.0 + em2
    g_ref[...] = jnp.where(
        col == 0, 1.0 / denom, jnp.where(col == 1, em2 / denom, 0.0))

    sel = ((col == idx1) | (col == idx2)).astype(jnp.int32)
    counts = jnp.sum(sel, axis=0, keepdims=True)
    pc = ((counts + BM - 1) // BM) * BM
    col1 = lax.broadcasted_iota(jnp.int32, (1, EP), 1)
    p = pc
    for sh in (1, 2, 4):
        rolled = jnp.concatenate(
            [jnp.zeros((1, sh), jnp.int32), p[:, :EP - sh]], axis=1)
        p = p + jnp.where(col1 >= sh, rolled, 0)
    off = p - pc
    inc = sel
    sh = 1
    while sh < T:
        inc = inc + _shift_down(inc, sh)
        sh *= 2
    rank = inc - sel
    dest = off + rank
    d1 = jnp.sum(jnp.where(col == idx1, dest, 0), axis=1, keepdims=True)
    d2 = jnp.sum(jnp.where(col == idx2, dest, 0), axis=1, keepdims=True)
    dest_ref[...] = jnp.where(col == 0, d1, jnp.where(col == 1, d2, 0))
    cnt_ref[...] = jnp.broadcast_to(counts, (8, EP))


@functools.cache
def _sc_mesh():
    return plsc.VectorSubcoreMesh(core_axis_name="c", subcore_axis_name="s")


def _sc_combine(os_hbm, d0_hbm, d1_hbm, out_hbm, idx_v, r0_v, r1_v, sem):
    chunk = r0_v.shape[0]
    per = T // NW
    wid = lax.axis_index("s") * NC + lax.axis_index("c")
    base = wid * per

    @pl.loop(0, per // chunk)
    def _(c):
        b = base + c * chunk
        pltpu.sync_copy(d0_hbm.at[pl.ds(b, chunk)], idx_v)
        pltpu.async_copy(os_hbm.at[idx_v], r0_v, sem).wait()
        pltpu.sync_copy(d1_hbm.at[pl.ds(b, chunk)], idx_v)
        pltpu.async_copy(os_hbm.at[idx_v], r1_v, sem).wait()

        @pl.loop(0, chunk)
        def _(r):
            @pl.loop(0, D // 16)
            def _(j):
                sl = (pl.ds(r, 1), pl.ds(j * 16, 16))
                r0_v.at[sl][...] = r0_v.at[sl][...] + r1_v.at[sl][...]

        pltpu.sync_copy(r0_v, out_hbm.at[pl.ds(b, chunk)])


def _gffn_kernel(se_ref, nu_ref, xb_ref, d01_ref, g01_ref,
                 w1_ref, b1_ref, w2_ref, b2_ref, o_ref,
                 acc_ref, xs_ref, gs_ref):
    hb = pl.program_id(0)
    s = pl.program_id(1)

    @pl.when(s < nu_ref[0])
    def _():
        row = pl.ds(s * BM, BM)

        @pl.when(hb == 0)
        def _():
            jrow = s * BM + lax.broadcasted_iota(jnp.int32, (BM, T), 0)
            d0b = jnp.broadcast_to(d01_ref[0:1, :], (BM, T))
            d1b = jnp.broadcast_to(d01_ref[1:2, :], (BM, T))
            c0 = d0b == jrow
            c1 = d1b == jrow
            pmat = jnp.where(c0 | c1, jnp.float32(1),
                             jnp.float32(0)).astype(jnp.bfloat16)
            xs = lax.dot_general(
                pmat, xb_ref[...], (((1,), (0,)), ((), ())),
                preferred_element_type=jnp.float32)
            xs_ref[row, :] = xs.astype(jnp.bfloat16)
            g0b = jnp.broadcast_to(g01_ref[0:1, :], (BM, T))
            g1b = jnp.broadcast_to(g01_ref[1:2, :], (BM, T))
            g = (jnp.sum(jnp.where(c0, g0b, 0.0), axis=1, keepdims=True)
                 + jnp.sum(jnp.where(c1, g1b, 0.0), axis=1, keepdims=True))
            gs_ref[row, :] = g

        xs = xs_ref[row, :]
        w1 = w1_ref[0].astype(jnp.bfloat16)
        h = lax.dot_general(
            xs, w1, (((1,), (0,)), ((), ())),
            preferred_element_type=jnp.float32)
        h = h + b1_ref[0]
        h = 0.5 * h * (1.0 + lax.erf(h * 0.7071067811865476))
        w2 = w2_ref[0].astype(jnp.bfloat16)
        po = lax.dot_general(
            h.astype(jnp.bfloat16), w2, (((1,), (0,)), ((), ())),
            preferred_element_type=jnp.float32)
        g = gs_ref[row, :]
        contrib = g * po

        @pl.when(hb == 0)
        def _():
            acc_ref[row, :] = contrib + g * b2_ref[0]

        @pl.when(hb > 0)
        def _():
            acc_ref[row, :] += contrib

        @pl.when(hb == NHB - 1)
        def _():
            o_ref[...] = acc_ref[row, :]


def kernel(x, W_route, b_route, W_noise, b_noise, fc1_w, fc1_b, fc2_w, fc2_b):
    i32 = jnp.int32
    x2 = x.reshape(T, D)
    w = jnp.zeros((D, 2 * EP), jnp.float32)
    w = w.at[:, :E].set(W_route).at[:, EP:EP + E].set(W_noise)
    bvec = jnp.zeros((2 * EP,), jnp.float32)
    bvec = bvec.at[:E].set(b_route).at[EP:EP + E].set(b_noise)
    bvec = jnp.broadcast_to(bvec[None, :], (8, 2 * EP))
    eps = jax.random.normal(jax.random.key(42), (B, S, E), dtype=jnp.float32)
    eps_p = jnp.zeros((T, EP), jnp.float32).at[:, :E].set(eps.reshape(T, E))

    g_tk, dest_tk, cnt = pl.pallas_call(
        _router_kernel,
        out_shape=(
            jax.ShapeDtypeStruct((T, EP), jnp.float32),
            jax.ShapeDtypeStruct((T, EP), i32),
            jax.ShapeDtypeStruct((8, EP), i32),
        ),
    )(x2, w, bvec, eps_p)

    counts8 = cnt[0, :E]
    tiles = (counts8 + BM - 1) // BM
    tstart = jnp.concatenate([jnp.zeros((1,), i32), jnp.cumsum(tiles)])
    nused = tstart[E:E + 1].astype(i32)
    s_iota = jnp.arange(NT, dtype=i32)
    step_expert = jnp.minimum(
        jnp.sum((s_iota[:, None] >= tstart[None, 1:]).astype(i32), axis=1),
        E - 1).astype(i32)

    dest2t = dest_tk[:, :K].T
    g2t = g_tk[:, :K].T
    d0 = dest2t[0]
    d1 = dest2t[1]
    xbf = x2.astype(jnp.bfloat16)
    d01 = jnp.zeros((8, T), i32).at[:K].set(dest2t)
    g01 = jnp.zeros((8, T), jnp.float32).at[:K].set(g2t)

    o_sorted = pl.pallas_call(
        _gffn_kernel,
        grid_spec=pltpu.PrefetchScalarGridSpec(
            num_scalar_prefetch=2,
            grid=(NHB, NT),
            in_specs=[
                pl.BlockSpec((T, D), lambda hb, s, se, nu: (0, 0)),
                pl.BlockSpec((8, T), lambda hb, s, se, nu: (0, 0)),
                pl.BlockSpec((8, T), lambda hb, s, se, nu: (0, 0)),
                pl.BlockSpec(
                    (1, D, HB),
                    lambda hb, s, se, nu:
                    (se[jnp.minimum(s, nu[0] - 1)], 0, hb)),
                pl.BlockSpec(
                    (1, 1, HB),
                    lambda hb, s, se, nu:
                    (se[jnp.minimum(s, nu[0] - 1)], 0, hb)),
                pl.BlockSpec(
                    (1, HB, D),
                    lambda hb, s, se, nu:
                    (se[jnp.minimum(s, nu[0] - 1)], hb, 0)),
                pl.BlockSpec(
                    (1, 1, D),
                    lambda hb, s, se, nu:
                    (se[jnp.minimum(s, nu[0] - 1)], 0, 0)),
            ],
            out_specs=pl.BlockSpec(
                (BM, D),
                lambda hb, s, se, nu: (jnp.where(hb == NHB - 1, s, 0), 0)),
            scratch_shapes=[
                pltpu.VMEM((L, D), jnp.float32),
                pltpu.VMEM((L, D), jnp.bfloat16),
                pltpu.VMEM((L, 1), jnp.float32),
            ],
        ),
        out_shape=jax.ShapeDtypeStruct((L, D), jnp.float32),
    )(step_expert, nused, xbf, d01, g01,
      fc1_w, fc1_b.reshape(E, 1, H), fc2_w, fc2_b.reshape(E, 1, D))

    out = pl.kernel(
        _sc_combine,
        out_type=jax.ShapeDtypeStruct((T, D), jnp.float32),
        mesh=_sc_mesh(),
        scratch_types=[
            pltpu.VMEM((32,), i32),
            pltpu.VMEM((32, D), jnp.float32),
            pltpu.VMEM((32, D), jnp.float32),
            pltpu.SemaphoreType.DMA,
        ],
    )(o_sorted, d0, d1)

    return out.reshape(B, S, D)


# speedup vs baseline: 1.3991x; 1.1316x over previous
import functools

import jax
import jax.numpy as jnp
from jax import lax
from jax.experimental import pallas as pl
from jax.experimental.pallas import tpu as pltpu
from jax.experimental.pallas import tpu_sc as plsc

B, S, D, H, E, K = 1, 2048, 1024, 4096, 8, 2
T = B * S
EP = 128
BM = 256
NT = (K * T) // BM + E
L = NT * BM
NHB = 2
HB = H // NHB
MT = T // BM

NC, NS = 2, 16
NW = NC * NS


def _shift_down(a, sh):
    return jnp.concatenate(
        [jnp.zeros((sh,) + a.shape[1:], a.dtype), a[:-sh]], axis=0)


def _router_kernel(x_ref, w_ref, b_ref, eps_ref, g_ref, dest_ref, cnt_ref):
    x = x_ref[...]
    w = w_ref[...]
    res = lax.dot_general(
        x.astype(jnp.bfloat16), w.astype(jnp.bfloat16),
        (((1,), (0,)), ((), ())),
        preferred_element_type=jnp.float32,
    )
    b = b_ref[0:1, :]
    logits = res[:, :EP] + b[:, :EP]
    nlogits = res[:, EP:] + b[:, EP:]
    sp = jnp.logaddexp(nlogits, 0.0)
    noisy = logits + eps_ref[...] * sp
    col = lax.broadcasted_iota(jnp.int32, (T, EP), 1)
    neg = jnp.float32(-jnp.inf)
    noisy = jnp.where(col < E, noisy, neg)
    m1 = jnp.max(noisy, axis=1, keepdims=True)
    idx1 = jnp.min(jnp.where(noisy == m1, col, EP), axis=1, keepdims=True)
    v2 = jnp.where(col == idx1, neg, noisy)
    m2 = jnp.max(v2, axis=1, keepdims=True)
    idx2 = jnp.min(jnp.where(v2 == m2, col, EP), axis=1, keepdims=True)
    em2 = jnp.exp(m2 - m1)
    denom = 1.0 + em2
    g_ref[...] = jnp.where(
        col == 0, 1.0 / denom, jnp.where(col == 1, em2 / denom, 0.0))

    sel = ((col == idx1) | (col == idx2)).astype(jnp.int32)
    counts = jnp.sum(sel, axis=0, keepdims=True)
    pc = ((counts + BM - 1) // BM) * BM
    col1 = lax.broadcasted_iota(jnp.int32, (1, EP), 1)
    p = pc
    for sh in (1, 2, 4):
        rolled = jnp.concatenate(
            [jnp.zeros((1, sh), jnp.int32), p[:, :EP - sh]], axis=1)
        p = p + jnp.where(col1 >= sh, rolled, 0)
    off = p - pc
    inc = sel
    sh = 1
    while sh < T:
        inc = inc + _shift_down(inc, sh)
        sh *= 2
    rank = inc - sel
    dest = off + rank
    d1 = jnp.sum(jnp.where(col == idx1, dest, 0), axis=1, keepdims=True)
    d2 = jnp.sum(jnp.where(col == idx2, dest, 0), axis=1, keepdims=True)
    dest_ref[...] = jnp.where(col == 0, d1, jnp.where(col == 1, d2, 0))
    cnt_ref[...] = jnp.broadcast_to(counts, (8, EP))


@functools.cache
def _sc_mesh():
    return plsc.VectorSubcoreMesh(core_axis_name="c", subcore_axis_name="s")


def _sc_combine(os_hbm, d0_hbm, d1_hbm, out_hbm, idx_v, r0_v, r1_v, sem):
    chunk = r0_v.shape[0]
    per = T // NW
    wid = lax.axis_index("s") * NC + lax.axis_index("c")
    base = wid * per

    @pl.loop(0, per // chunk)
    def _(c):
        b = base + c * chunk
        pltpu.sync_copy(d0_hbm.at[pl.ds(b, chunk)], idx_v)
        pltpu.async_copy(os_hbm.at[idx_v], r0_v, sem).wait()
        pltpu.sync_copy(d1_hbm.at[pl.ds(b, chunk)], idx_v)
        pltpu.async_copy(os_hbm.at[idx_v], r1_v, sem).wait()

        @pl.loop(0, chunk)
        def _(r):
            @pl.loop(0, D // 16)
            def _(j):
                sl = (pl.ds(r, 1), pl.ds(j * 16, 16))
                r0_v.at[sl][...] = r0_v.at[sl][...] + r1_v.at[sl][...]

        pltpu.sync_copy(r0_v, out_hbm.at[pl.ds(b, chunk)])


def _gffn_kernel(tiles_ref, tbase_ref, xb_ref, d01_ref, g01_ref,
                 w1_ref, b1_ref, w2_ref, b2_ref, o_ref,
                 acc_ref, xs_ref, gs_ref):
    e = pl.program_id(0)
    hb = pl.program_id(1)
    i = pl.program_id(2)

    @pl.when(i < tiles_ref[e])
    def _():
        gt = tbase_ref[e] + i
        row = pl.ds(i * BM, BM)

        @pl.when(hb == 0)
        def _():
            jrow = gt * BM + lax.broadcasted_iota(jnp.int32, (BM, T), 0)
            d0b = jnp.broadcast_to(d01_ref[0:1, :], (BM, T))
            d1b = jnp.broadcast_to(d01_ref[1:2, :], (BM, T))
            c0 = d0b == jrow
            c1 = d1b == jrow
            pmat = jnp.where(c0 | c1, jnp.float32(1),
                             jnp.float32(0)).astype(jnp.bfloat16)
            xs = lax.dot_general(
                pmat, xb_ref[...], (((1,), (0,)), ((), ())),
                preferred_element_type=jnp.float32)
            xs_ref[row, :] = xs.astype(jnp.bfloat16)
            g0b = jnp.broadcast_to(g01_ref[0:1, :], (BM, T))
            g1b = jnp.broadcast_to(g01_ref[1:2, :], (BM, T))
            g = (jnp.sum(jnp.where(c0, g0b, 0.0), axis=1, keepdims=True)
                 + jnp.sum(jnp.where(c1, g1b, 0.0), axis=1, keepdims=True))
            gs_ref[row, :] = g

        xs = xs_ref[row, :]
        w1 = w1_ref[0].astype(jnp.bfloat16)
        h = lax.dot_general(
            xs, w1, (((1,), (0,)), ((), ())),
            preferred_element_type=jnp.float32)
        h = h + b1_ref[0]
        h = 0.5 * h * (1.0 + lax.erf(h * 0.7071067811865476))
        w2 = w2_ref[0].astype(jnp.bfloat16)
        po = lax.dot_general(
            h.astype(jnp.bfloat16), w2, (((1,), (0,)), ((), ())),
            preferred_element_type=jnp.float32)
        g = gs_ref[row, :]
        contrib = g * po

        @pl.when(hb == 0)
        def _():
            acc_ref[row, :] = contrib + g * b2_ref[0]

        @pl.when(hb == NHB - 1)
        def _():
            o_ref[...] = acc_ref[row, :] + contrib


def kernel(x, W_route, b_route, W_noise, b_noise, fc1_w, fc1_b, fc2_w, fc2_b):
    i32 = jnp.int32
    x2 = x.reshape(T, D)
    w = jnp.zeros((D, 2 * EP), jnp.float32)
    w = w.at[:, :E].set(W_route).at[:, EP:EP + E].set(W_noise)
    bvec = jnp.zeros((2 * EP,), jnp.float32)
    bvec = bvec.at[:E].set(b_route).at[EP:EP + E].set(b_noise)
    bvec = jnp.broadcast_to(bvec[None, :], (8, 2 * EP))
    eps = jax.random.normal(jax.random.key(42), (B, S, E), dtype=jnp.float32)
    eps_p = jnp.zeros((T, EP), jnp.float32).at[:, :E].set(eps.reshape(T, E))

    g_tk, dest_tk, cnt = pl.pallas_call(
        _router_kernel,
        out_shape=(
            jax.ShapeDtypeStruct((T, EP), jnp.float32),
            jax.ShapeDtypeStruct((T, EP), i32),
            jax.ShapeDtypeStruct((8, EP), i32),
        ),
    )(x2, w, bvec, eps_p)

    counts8 = cnt[0, :E]
    tiles = (counts8 + BM - 1) // BM
    tstart = jnp.concatenate([jnp.zeros((1,), i32), jnp.cumsum(tiles)])
    tbase = tstart[:E].astype(i32)
    tiles = tiles.astype(i32)

    dest2t = dest_tk[:, :K].T
    g2t = g_tk[:, :K].T
    d0 = dest2t[0]
    d1 = dest2t[1]
    xbf = x2.astype(jnp.bfloat16)
    d01 = jnp.zeros((8, T), i32).at[:K].set(dest2t)
    g01 = jnp.zeros((8, T), jnp.float32).at[:K].set(g2t)

    o_sorted = pl.pallas_call(
        _gffn_kernel,
        grid_spec=pltpu.PrefetchScalarGridSpec(
            num_scalar_prefetch=2,
            grid=(E, NHB, MT),
            in_specs=[
                pl.BlockSpec((T, D), lambda e, hb, i, tl, tb: (0, 0)),
                pl.BlockSpec((8, T), lambda e, hb, i, tl, tb: (0, 0)),
                pl.BlockSpec((8, T), lambda e, hb, i, tl, tb: (0, 0)),
                pl.BlockSpec(
                    (1, D, HB),
                    lambda e, hb, i, tl, tb: (e, 0, hb)),
                pl.BlockSpec(
                    (1, 1, HB),
                    lambda e, hb, i, tl, tb: (e, 0, hb)),
                pl.BlockSpec(
                    (1, HB, D),
                    lambda e, hb, i, tl, tb: (e, hb, 0)),
                pl.BlockSpec(
                    (1, 1, D),
                    lambda e, hb, i, tl, tb: (e, 0, 0)),
            ],
            out_specs=pl.BlockSpec(
                (BM, D),
                lambda e, hb, i, tl, tb: (jnp.minimum(
                    tb[e] + jnp.minimum(i, jnp.maximum(tl[e] - 1, 0)),
                    NT - 1), 0)),
            scratch_shapes=[
                pltpu.VMEM((MT * BM, D), jnp.float32),
                pltpu.VMEM((MT * BM, D), jnp.bfloat16),
                pltpu.VMEM((MT * BM, 1), jnp.float32),
            ],
        ),
        out_shape=jax.ShapeDtypeStruct((L, D), jnp.float32),
    )(tiles, tbase, xbf, d01, g01,
      fc1_w, fc1_b.reshape(E, 1, H), fc2_w, fc2_b.reshape(E, 1, D))

    out = pl.kernel(
        _sc_combine,
        out_type=jax.ShapeDtypeStruct((T, D), jnp.float32),
        mesh=_sc_mesh(),
        scratch_types=[
            pltpu.VMEM((32,), i32),
            pltpu.VMEM((32, D), jnp.float32),
            pltpu.VMEM((32, D), jnp.float32),
            pltpu.SemaphoreType.DMA,
        ],
    )(o_sorted, d0, d1)

    return out.reshape(B, S, D)
